# Initial kernel scaffold; baseline (speedup 1.0000x reference)
import jax
import jax.numpy as jnp
from jax.experimental import pallas as pl


def kernel(x, W1, b1, W2, b2):
    raise NotImplementedError("write your pallas kernel here")


# speedup vs baseline: 6.6597x; 6.6597x over previous
import functools

import jax
import jax.numpy as jnp
from jax.experimental import pallas as pl

_K = 16


def _embed_kernel(x_ref, w1_ref, b1_ref, w2_ref, b2_ref, yn_ref):
    h = jnp.dot(x_ref[...], w1_ref[...], preferred_element_type=jnp.float32)
    h = h + b1_ref[...]
    h = jnp.where(h >= 0, h, 0.01 * h)
    y = jnp.dot(h, w2_ref[...], preferred_element_type=jnp.float32)
    y = y + b2_ref[...]
    norm = jnp.sqrt(jnp.sum(y * y, axis=1, keepdims=True))
    yn_ref[...] = y / jnp.maximum(norm, 1e-12)


def _topk_kernel(q_ref, keys_ref, idx_ref, *, block_r: int, n: int):
    i = pl.program_id(0)
    sim = jax.lax.dot_general(
        q_ref[...], keys_ref[...],
        (((1,), (1,)), ((), ())),
        preferred_element_type=jnp.float32,
    )
    col = jax.lax.broadcasted_iota(jnp.int32, (block_r, n), 1)
    row = jax.lax.broadcasted_iota(jnp.int32, (block_r, n), 0) + i * block_r
    sim = jnp.where(col == row, -1.0, sim)
    for k in range(_K):
        a = jnp.argmax(sim, axis=1).astype(jnp.int32)
        idx_ref[:, k] = a
        sim = jnp.where(col == a[:, None], -3.0, sim)


def kernel(x, W1, b1, W2, b2):
    n, in_dim = x.shape
    hidden = W1.shape[1]
    block_r = 400 if n % 400 == 0 else n

    yn = pl.pallas_call(
        _embed_kernel,
        grid=(n // block_r,),
        in_specs=[
            pl.BlockSpec((block_r, in_dim), lambda i: (i, 0)),
            pl.BlockSpec((in_dim, hidden), lambda i: (0, 0)),
            pl.BlockSpec((1, hidden), lambda i: (0, 0)),
            pl.BlockSpec((hidden, in_dim), lambda i: (0, 0)),
            pl.BlockSpec((1, in_dim), lambda i: (0, 0)),
        ],
        out_specs=pl.BlockSpec((block_r, in_dim), lambda i: (i, 0)),
        out_shape=jax.ShapeDtypeStruct((n, in_dim), jnp.float32),
    )(x, W1, b1.reshape(1, hidden), W2, b2.reshape(1, in_dim))

    idx = pl.pallas_call(
        functools.partial(_topk_kernel, block_r=block_r, n=n),
        grid=(n // block_r,),
        in_specs=[
            pl.BlockSpec((block_r, in_dim), lambda i: (i, 0)),
            pl.BlockSpec((n, in_dim), lambda i: (0, 0)),
        ],
        out_specs=pl.BlockSpec((block_r, _K), lambda i: (i, 0)),
        out_shape=jax.ShapeDtypeStruct((n, _K), jnp.int32),
    )(yn, yn)

    src = jnp.repeat(jnp.arange(n, dtype=jnp.int32), _K)
    dst = idx.reshape(-1)
    return src, dst


# speedup vs baseline: 9.3774x; 1.4081x over previous
import functools

import jax
import jax.numpy as jnp
from jax import lax
from jax.experimental import pallas as pl
from jax.experimental.pallas import tpu as pltpu
from jax.experimental.pallas import tpu_sc as plsc

_K = 16
_L = 128
_BATCH = 16
_NW = 32
_W = _K * _L


def _embed_kernel(x_ref, w1_ref, b1_ref, w2_ref, b2_ref, yn_ref):
    h = jnp.dot(x_ref[...], w1_ref[...], preferred_element_type=jnp.float32)
    h = h + b1_ref[...]
    h = jnp.where(h >= 0, h, 0.01 * h)
    y = jnp.dot(h, w2_ref[...], preferred_element_type=jnp.float32)
    y = y + b2_ref[...]
    norm = jnp.sqrt(jnp.sum(y * y, axis=1, keepdims=True))
    yn_ref[...] = y / jnp.maximum(norm, 1e-12)


def _strip_kernel(q_ref, keys_ref, sim_ref, gids_ref, *,
                  block_r: int, n: int, npad: int):
    i = pl.program_id(0)
    sim = lax.dot_general(
        q_ref[...], keys_ref[...],
        (((1,), (1,)), ((), ())),
        preferred_element_type=jnp.float32,
    )
    col = lax.broadcasted_iota(jnp.int32, (block_r, npad), 1)
    row = lax.broadcasted_iota(jnp.int32, (block_r, npad), 0) + i * block_r
    sim = jnp.where(col >= n, -3.0, sim)
    sim = jnp.where(col == row, -1.0, sim)
    sim_ref[...] = sim

    g = npad // _L
    gm = jnp.max(sim.reshape(block_r, g, _L), axis=2)
    gm = jnp.concatenate(
        [gm, jnp.full((block_r, 1), -3.0, jnp.float32)], axis=1)
    colg = lax.broadcasted_iota(jnp.int32, (block_r, g + 1), 1)
    for k in range(_K):
        a = jnp.argmax(gm, axis=1).astype(jnp.int32)
        gids_ref[:, k] = a
        gm = jnp.where(colg == a[:, None], -4.0, gm)


def _sc_gather_body(sim_ref, gids_ref, out_ref, gidsv, idx_a, idx_b,
                    candv, sem, *, g: int, n_batches: int, iters: int):
    cid = lax.axis_index("c")
    sid = lax.axis_index("s")
    wid = sid * 2 + cid

    def batch_body(it, carry):
        b = wid + it * _NW

        @pl.when(b < n_batches)
        def _():
            r0 = b * _BATCH
            pltpu.sync_copy(gids_ref.at[pl.ds(b * _BATCH * _K, _BATCH * _K)],
                            gidsv)
            for m in range(_BATCH):
                gv = gidsv[pl.ds(m * _K, _K)]
                rowbase = (r0 + m) * g
                if m < 8:
                    idx_a[pl.ds(m * _K, _K)] = gv + rowbase
                else:
                    idx_b[pl.ds((m - 8) * _K, _K)] = gv + rowbase
            cp1 = pltpu.async_copy(sim_ref.at[idx_a],
                                   candv.at[pl.ds(0, _L)], sem)
            cp2 = pltpu.async_copy(sim_ref.at[idx_b],
                                   candv.at[pl.ds(_L, _L)], sem)
            cp1.wait()
            cp2.wait()
            pltpu.sync_copy(candv,
                            out_ref.at[pl.ds(b * _BATCH * _K, _BATCH * _K)])

        return carry

    lax.fori_loop(0, iters, batch_body, None)


def _select_kernel(cand_ref, gids_ref, idx_ref, *, block_r: int):
    cand = cand_ref[...]
    gids = gids_ref[...]
    colp = lax.broadcasted_iota(jnp.int32, (block_r, _W), 1)
    gj = lax.broadcasted_iota(jnp.int32, (block_r, _K), 1)
    for k in range(_K):
        a = jnp.argmax(cand, axis=1).astype(jnp.int32)
        j = a >> 7
        gsel = jnp.sum(jnp.where(gj == j[:, None], gids, 0), axis=1)
        idx_ref[:, k] = gsel * _L + (a & (_L - 1))
        cand = jnp.where(colp == a[:, None], -3.0, cand)


def kernel(x, W1, b1, W2, b2):
    n, in_dim = x.shape
    hidden = W1.shape[1]
    block_r = 400
    npad = ((n // _L) + 1) * _L if n % _L else n
    g = npad // _L

    yn = pl.pallas_call(
        _embed_kernel,
        grid=(n // block_r,),
        in_specs=[
            pl.BlockSpec((block_r, in_dim), lambda i: (i, 0)),
            pl.BlockSpec((in_dim, hidden), lambda i: (0, 0)),
            pl.BlockSpec((1, hidden), lambda i: (0, 0)),
            pl.BlockSpec((hidden, in_dim), lambda i: (0, 0)),
            pl.BlockSpec((1, in_dim), lambda i: (0, 0)),
        ],
        out_specs=pl.BlockSpec((block_r, in_dim), lambda i: (i, 0)),
        out_shape=jax.ShapeDtypeStruct((n, in_dim), jnp.float32),
    )(x, W1, b1.reshape(1, hidden), W2, b2.reshape(1, in_dim))

    keys = jnp.pad(yn, ((0, npad - n), (0, 0)))

    sim, gids = pl.pallas_call(
        functools.partial(_strip_kernel, block_r=block_r, n=n, npad=npad),
        grid=(n // block_r,),
        in_specs=[
            pl.BlockSpec((block_r, in_dim), lambda i: (i, 0)),
            pl.BlockSpec((npad, in_dim), lambda i: (0, 0)),
        ],
        out_specs=[
            pl.BlockSpec((block_r, npad), lambda i: (i, 0)),
            pl.BlockSpec((block_r, _K), lambda i: (i, 0)),
        ],
        out_shape=[
            jax.ShapeDtypeStruct((n, npad), jnp.float32),
            jax.ShapeDtypeStruct((n, _K), jnp.int32),
        ],
    )(yn, keys)

    n_batches = n // _BATCH
    iters = (n_batches + _NW - 1) // _NW
    mesh = plsc.VectorSubcoreMesh(core_axis_name="c", subcore_axis_name="s")
    sc_fn = pl.kernel(
        functools.partial(_sc_gather_body, g=g,
                          n_batches=n_batches, iters=iters),
        out_type=jax.ShapeDtypeStruct((n * _K, _L), jnp.float32),
        mesh=mesh,
        scratch_types=[
            pltpu.VMEM((_BATCH * _K,), jnp.int32),
            pltpu.VMEM((_L,), jnp.int32),
            pltpu.VMEM((_L,), jnp.int32),
            pltpu.VMEM((_BATCH * _K, _L), jnp.float32),
            pltpu.SemaphoreType.DMA,
        ],
    )
    cand = sc_fn(sim.reshape(n * g, _L), gids.reshape(-1))

    idx = pl.pallas_call(
        functools.partial(_select_kernel, block_r=block_r),
        grid=(n // block_r,),
        in_specs=[
            pl.BlockSpec((block_r, _W), lambda i: (i, 0)),
            pl.BlockSpec((block_r, _K), lambda i: (i, 0)),
        ],
        out_specs=pl.BlockSpec((block_r, _K), lambda i: (i, 0)),
        out_shape=jax.ShapeDtypeStruct((n, _K), jnp.int32),
    )(cand.reshape(n, _W), gids)

    src = jnp.repeat(jnp.arange(n, dtype=jnp.int32), _K)
    dst = idx.reshape(-1)
    return src, dst


# speedup vs baseline: 10.4205x; 1.1112x over previous
import functools

import jax
import jax.numpy as jnp
from jax import lax
from jax.experimental import pallas as pl
from jax.experimental.pallas import tpu as pltpu
from jax.experimental.pallas import tpu_sc as plsc

_K = 16
_L = 128
_BATCH = 16
_NW = 32
_W = _K * _L


def _embed_kernel(x_ref, w1_ref, b1_ref, w2_ref, b2_ref, yn_ref):
    h = jnp.dot(x_ref[...], w1_ref[...], preferred_element_type=jnp.float32)
    h = h + b1_ref[...]
    h = jnp.where(h >= 0, h, 0.01 * h)
    y = jnp.dot(h, w2_ref[...], preferred_element_type=jnp.float32)
    y = y + b2_ref[...]
    norm = jnp.sqrt(jnp.sum(y * y, axis=1, keepdims=True))
    yn_ref[...] = y / jnp.maximum(norm, 1e-12)


def _strip_kernel(q_ref, keys_ref, sim_ref, gids_ref, *,
                  block_r: int, n: int, npad: int):
    i = pl.program_id(0)
    sim = lax.dot_general(
        q_ref[...], keys_ref[...],
        (((1,), (1,)), ((), ())),
        preferred_element_type=jnp.float32,
    )
    col = lax.broadcasted_iota(jnp.int32, (block_r, npad), 1)
    row = lax.broadcasted_iota(jnp.int32, (block_r, npad), 0) + i * block_r
    sim = jnp.where(col >= n, -3.0, sim)
    sim = jnp.where(col == row, -1.0, sim)
    sim_ref[...] = sim

    g = npad // _L
    gm = jnp.max(sim.reshape(block_r, g, _L), axis=2)
    gm = jnp.concatenate(
        [gm, jnp.full((block_r, 1), -3.0, jnp.float32)], axis=1)
    colg = lax.broadcasted_iota(jnp.int32, (block_r, g + 1), 1)
    for k in range(_K):
        a = jnp.argmax(gm, axis=1).astype(jnp.int32)
        gids_ref[:, k] = a
        gm = jnp.where(colg == a[:, None], -4.0, gm)


def _sc_gather_body(sim_ref, gids_ref, out_ref, gidsv, idx_a, idx_b,
                    candv, sem, *, g: int, n_batches: int, iters: int):
    cid = lax.axis_index("c")
    sid = lax.axis_index("s")
    wid = sid * 2 + cid

    def batch_body(it, carry):
        b = wid + it * _NW

        @pl.when(b < n_batches)
        def _():
            r0 = b * _BATCH
            pltpu.sync_copy(gids_ref.at[pl.ds(b * _BATCH * _K, _BATCH * _K)],
                            gidsv)
            for m in range(_BATCH):
                gv = gidsv[pl.ds(m * _K, _K)]
                rowbase = (r0 + m) * g
                if m < 8:
                    idx_a[pl.ds(m * _K, _K)] = gv + rowbase
                else:
                    idx_b[pl.ds((m - 8) * _K, _K)] = gv + rowbase
            cp1 = pltpu.async_copy(sim_ref.at[idx_a],
                                   candv.at[pl.ds(0, _L)], sem)
            cp2 = pltpu.async_copy(sim_ref.at[idx_b],
                                   candv.at[pl.ds(_L, _L)], sem)
            cp1.wait()
            cp2.wait()
            pltpu.sync_copy(candv,
                            out_ref.at[pl.ds(b * _BATCH * _K, _BATCH * _K)])

        return carry

    lax.fori_loop(0, iters, batch_body, None)


def _select_kernel(cand_ref, gids_ref, idx_ref, *, block_r: int):
    cand = cand_ref[...]
    gids = gids_ref[...]
    colp = lax.broadcasted_iota(jnp.int32, (block_r, _W), 1)
    gcol = (jnp.repeat(gids, _L, axis=1) * _L) + (colp & (_L - 1))
    big = jnp.int32(2 ** 30)
    for k in range(_K):
        m = jnp.max(cand, axis=1)
        c = jnp.min(jnp.where(cand == m[:, None], gcol, big), axis=1)
        idx_ref[:, k] = c
        cand = jnp.where(gcol == c[:, None], -3.0, cand)


def kernel(x, W1, b1, W2, b2):
    n, in_dim = x.shape
    hidden = W1.shape[1]
    block_r = 400
    npad = ((n // _L) + 1) * _L if n % _L else n
    g = npad // _L

    yn = pl.pallas_call(
        _embed_kernel,
        grid=(n // block_r,),
        in_specs=[
            pl.BlockSpec((block_r, in_dim), lambda i: (i, 0)),
            pl.BlockSpec((in_dim, hidden), lambda i: (0, 0)),
            pl.BlockSpec((1, hidden), lambda i: (0, 0)),
            pl.BlockSpec((hidden, in_dim), lambda i: (0, 0)),
            pl.BlockSpec((1, in_dim), lambda i: (0, 0)),
        ],
        out_specs=pl.BlockSpec((block_r, in_dim), lambda i: (i, 0)),
        out_shape=jax.ShapeDtypeStruct((n, in_dim), jnp.float32),
    )(x, W1, b1.reshape(1, hidden), W2, b2.reshape(1, in_dim))

    keys = jnp.pad(yn, ((0, npad - n), (0, 0)))

    sim, gids = pl.pallas_call(
        functools.partial(_strip_kernel, block_r=block_r, n=n, npad=npad),
        grid=(n // block_r,),
        in_specs=[
            pl.BlockSpec((block_r, in_dim), lambda i: (i, 0)),
            pl.BlockSpec((npad, in_dim), lambda i: (0, 0)),
        ],
        out_specs=[
            pl.BlockSpec((block_r, npad), lambda i: (i, 0)),
            pl.BlockSpec((block_r, _K), lambda i: (i, 0)),
        ],
        out_shape=[
            jax.ShapeDtypeStruct((n, npad), jnp.float32),
            jax.ShapeDtypeStruct((n, _K), jnp.int32),
        ],
    )(yn, keys)

    n_batches = n // _BATCH
    iters = (n_batches + _NW - 1) // _NW
    mesh = plsc.VectorSubcoreMesh(core_axis_name="c", subcore_axis_name="s")
    sc_fn = pl.kernel(
        functools.partial(_sc_gather_body, g=g,
                          n_batches=n_batches, iters=iters),
        out_type=jax.ShapeDtypeStruct((n * _K, _L), jnp.float32),
        mesh=mesh,
        scratch_types=[
            pltpu.VMEM((_BATCH * _K,), jnp.int32),
            pltpu.VMEM((_L,), jnp.int32),
            pltpu.VMEM((_L,), jnp.int32),
            pltpu.VMEM((_BATCH * _K, _L), jnp.float32),
            pltpu.SemaphoreType.DMA,
        ],
    )
    cand = sc_fn(sim.reshape(n * g, _L), gids.reshape(-1))

    idx = pl.pallas_call(
        functools.partial(_select_kernel, block_r=block_r),
        grid=(n // block_r,),
        in_specs=[
            pl.BlockSpec((block_r, _W), lambda i: (i, 0)),
            pl.BlockSpec((block_r, _K), lambda i: (i, 0)),
        ],
        out_specs=pl.BlockSpec((block_r, _K), lambda i: (i, 0)),
        out_shape=jax.ShapeDtypeStruct((n, _K), jnp.int32),
    )(cand.reshape(n, _W), gids)

    src = jnp.repeat(jnp.arange(n, dtype=jnp.int32), _K)
    dst = idx.reshape(-1)
    return src, dst


# speedup vs baseline: 10.5596x; 1.0134x over previous
import functools

import jax
import jax.numpy as jnp
from jax import lax
from jax.experimental import pallas as pl
from jax.experimental.pallas import tpu as pltpu
from jax.experimental.pallas import tpu_sc as plsc

_K = 16
_L = 128
_BATCH = 16
_NW = 32
_W = _K * _L


def _embed_kernel(x_ref, w1_ref, b1_ref, w2_ref, b2_ref, yn_ref):
    h = jnp.dot(x_ref[...], w1_ref[...], preferred_element_type=jnp.float32)
    h = h + b1_ref[...]
    h = jnp.where(h >= 0, h, 0.01 * h)
    y = jnp.dot(h, w2_ref[...], preferred_element_type=jnp.float32)
    y = y + b2_ref[...]
    norm = jnp.sqrt(jnp.sum(y * y, axis=1, keepdims=True))
    yn_ref[...] = y / jnp.maximum(norm, 1e-12)


def _strip_kernel(q_ref, keys_ref, sim_ref, gids_ref, *,
                  block_r: int, n: int, npad: int, row0: int = 0):
    i = pl.program_id(0)
    sim = lax.dot_general(
        q_ref[...], keys_ref[...],
        (((1,), (1,)), ((), ())),
        preferred_element_type=jnp.float32,
    )
    col = lax.broadcasted_iota(jnp.int32, (block_r, npad), 1)
    row = (lax.broadcasted_iota(jnp.int32, (block_r, npad), 0)
           + i * block_r + row0)
    sim = jnp.where(col >= n, -3.0, sim)
    sim = jnp.where(col == row, -1.0, sim)
    sim_ref[...] = sim

    g = npad // _L
    gm = jnp.max(sim.reshape(block_r, g, _L), axis=2)
    gm = jnp.concatenate(
        [gm, jnp.full((block_r, 1), -3.0, jnp.float32)], axis=1)
    colg = lax.broadcasted_iota(jnp.int32, (block_r, g + 1), 1)
    for k in range(_K):
        a = jnp.argmax(gm, axis=1).astype(jnp.int32)
        gids_ref[:, k] = a
        gm = jnp.where(colg == a[:, None], -4.0, gm)


def _sc_gather_body(sim_ref, gids_ref, out_ref, gidsv, idx_a, idx_b,
                    candv, sem, *, g: int, n_batches: int, iters: int):
    cid = lax.axis_index("c")
    sid = lax.axis_index("s")
    wid = sid * 2 + cid

    def batch_body(it, carry):
        b = wid + it * _NW

        @pl.when(b < n_batches)
        def _():
            r0 = b * _BATCH
            pltpu.sync_copy(gids_ref.at[pl.ds(b * _BATCH * _K, _BATCH * _K)],
                            gidsv)
            for m in range(_BATCH):
                gv = gidsv[pl.ds(m * _K, _K)]
                rowbase = (r0 + m) * g
                if m < 8:
                    idx_a[pl.ds(m * _K, _K)] = gv + rowbase
                else:
                    idx_b[pl.ds((m - 8) * _K, _K)] = gv + rowbase
            half = _BATCH * _K // 2
            cp1 = pltpu.async_copy(sim_ref.at[idx_a],
                                   candv.at[pl.ds(0, half)], sem)
            cp2 = pltpu.async_copy(sim_ref.at[idx_b],
                                   candv.at[pl.ds(half, half)], sem)
            cp1.wait()
            cp2.wait()
            pltpu.sync_copy(candv,
                            out_ref.at[pl.ds(b * _BATCH * _K, _BATCH * _K)])

        return carry

    lax.fori_loop(0, iters, batch_body, None)


def _select_kernel(cand_ref, gids_ref, idx_ref, *, block_r: int):
    cand = cand_ref[...]
    gids = gids_ref[...]
    colp = lax.broadcasted_iota(jnp.int32, (block_r, _W), 1)
    gcol = (jnp.repeat(gids, _L, axis=1) * _L) + (colp & (_L - 1))
    big = jnp.int32(2 ** 30)
    for k in range(_K):
        m = jnp.max(cand, axis=1)
        c = jnp.min(jnp.where(cand == m[:, None], gcol, big), axis=1)
        idx_ref[:, k] = c
        cand = jnp.where(gcol == c[:, None], -3.0, cand)


def kernel(x, W1, b1, W2, b2):
    n, in_dim = x.shape
    hidden = W1.shape[1]
    block_r = 400
    npad = ((n // _L) + 1) * _L if n % _L else n
    g = npad // _L

    yn = pl.pallas_call(
        _embed_kernel,
        grid=(n // block_r,),
        in_specs=[
            pl.BlockSpec((block_r, in_dim), lambda i: (i, 0)),
            pl.BlockSpec((in_dim, hidden), lambda i: (0, 0)),
            pl.BlockSpec((1, hidden), lambda i: (0, 0)),
            pl.BlockSpec((hidden, in_dim), lambda i: (0, 0)),
            pl.BlockSpec((1, in_dim), lambda i: (0, 0)),
        ],
        out_specs=pl.BlockSpec((block_r, in_dim), lambda i: (i, 0)),
        out_shape=jax.ShapeDtypeStruct((n, in_dim), jnp.float32),
    )(x, W1, b1.reshape(1, hidden), W2, b2.reshape(1, in_dim))

    keys = jnp.pad(yn, ((0, npad - n), (0, 0)))

    if n == 10000:
        splits = [(0, 4800), (4800, 5200)]
    else:
        splits = [(0, n)]

    mesh = plsc.VectorSubcoreMesh(core_axis_name="c", subcore_axis_name="s")

    def strip_half(row0, nh):
        return pl.pallas_call(
            functools.partial(_strip_kernel, block_r=block_r, n=n,
                              npad=npad, row0=row0),
            grid=(nh // block_r,),
            in_specs=[
                pl.BlockSpec((block_r, in_dim),
                             lambda i: (i, 0)),
                pl.BlockSpec((npad, in_dim), lambda i: (0, 0)),
            ],
            out_specs=[
                pl.BlockSpec((block_r, npad), lambda i: (i, 0)),
                pl.BlockSpec((block_r, _K), lambda i: (i, 0)),
            ],
            out_shape=[
                jax.ShapeDtypeStruct((nh, npad), jnp.float32),
                jax.ShapeDtypeStruct((nh, _K), jnp.int32),
            ],
        )(lax.dynamic_slice_in_dim(yn, row0, nh, 0), keys)

    def gather_half(sim_h, gids_h, nh):
        n_batches = nh // _BATCH
        iters = (n_batches + _NW - 1) // _NW
        sc_fn = pl.kernel(
            functools.partial(_sc_gather_body, g=g,
                              n_batches=n_batches, iters=iters),
            out_type=jax.ShapeDtypeStruct((nh * _K, _L), jnp.float32),
            mesh=mesh,
            scratch_types=[
                pltpu.VMEM((_BATCH * _K,), jnp.int32),
                pltpu.VMEM((_BATCH * _K // 2,), jnp.int32),
                pltpu.VMEM((_BATCH * _K // 2,), jnp.int32),
                pltpu.VMEM((_BATCH * _K, _L), jnp.float32),
                pltpu.SemaphoreType.DMA,
            ],
        )
        return sc_fn(sim_h.reshape(nh * g, _L), gids_h.reshape(-1))

    def select_half(cand_h, gids_h, nh):
        return pl.pallas_call(
            functools.partial(_select_kernel, block_r=block_r),
            grid=(nh // block_r,),
            in_specs=[
                pl.BlockSpec((block_r, _W), lambda i: (i, 0)),
                pl.BlockSpec((block_r, _K), lambda i: (i, 0)),
            ],
            out_specs=pl.BlockSpec((block_r, _K), lambda i: (i, 0)),
            out_shape=jax.ShapeDtypeStruct((nh, _K), jnp.int32),
        )(cand_h.reshape(nh, _W), gids_h)

    strips = [strip_half(r0, nh) for r0, nh in splits]
    cands = [gather_half(s, gg, nh)
             for (s, gg), (_, nh) in zip(strips, splits)]
    idxs = [select_half(c, gg, nh)
            for c, (_, gg), (_, nh) in zip(cands, strips, splits)]
    idx = jnp.concatenate(idxs, axis=0) if len(idxs) > 1 else idxs[0]

    src = jnp.repeat(jnp.arange(n, dtype=jnp.int32), _K)
    dst = idx.reshape(-1)
    return src, dst
